# baseline (speedup 1.0000x reference)
import functools

import jax
import jax.numpy as jnp
from jax.experimental import pallas as pl
from jax.experimental.pallas import tpu as pltpu

_BATCH = 4
_BLK_T = 512


def _ffn_block_kernel(x_ref, w1t_ref, w2t_ref, b2_ref, out_ref):
    x = x_ref[...].astype(jnp.bfloat16)
    h = jnp.dot(x, w1t_ref[...].astype(jnp.bfloat16),
                preferred_element_type=jnp.float32)
    h = 0.5 * h * (1.0 + jax.lax.erf(h * 0.7071067811865476))
    y = jnp.dot(h.astype(jnp.bfloat16), w2t_ref[...].astype(jnp.bfloat16),
                preferred_element_type=jnp.float32)
    y = y + b2_ref[...]
    out_ref[...] = jnp.broadcast_to(y[None], (_BATCH,) + y.shape)


@jax.jit
def _run(table, W1t, W2t, b2row):
    t_static, d = table.shape
    n_blocks = t_static // _BLK_T
    return pl.pallas_call(
        _ffn_block_kernel,
        grid=(n_blocks,),
        in_specs=[
            pl.BlockSpec((_BLK_T, d), lambda i: (i, 0)),
            pl.BlockSpec((d, d), lambda i: (0, 0)),
            pl.BlockSpec((d, d), lambda i: (0, 0)),
            pl.BlockSpec((1, d), lambda i: (0, 0)),
        ],
        out_specs=pl.BlockSpec((_BATCH, _BLK_T, d), lambda i: (0, i, 0)),
        out_shape=jax.ShapeDtypeStruct((_BATCH, t_static, d), jnp.float32),
        compiler_params=pltpu.CompilerParams(
            dimension_semantics=("arbitrary",),
        ),
    )(table, W1t, W2t, b2row)


def kernel(b, t, table, W1, W2, b2):
    return _run(table, W1.T, W2.T, b2.reshape(1, -1))


# speedup vs baseline: 1.0647x; 1.0647x over previous
import jax
import jax.numpy as jnp
from jax.experimental import pallas as pl
from jax.experimental.pallas import tpu as pltpu

_BATCH = 4
_BLK_T = 1024


def _ffn_block_kernel(x_ref, w1t_ref, w2t_ref, b2_ref, out_ref):
    x = x_ref[...]
    h = jnp.dot(x, w1t_ref[...], preferred_element_type=jnp.float32)
    h = 0.5 * h * (1.0 + jax.lax.erf(h * 0.7071067811865476))
    y = jnp.dot(h, w2t_ref[...], preferred_element_type=jnp.float32)
    y = y + b2_ref[...]
    out_ref[...] = jnp.broadcast_to(y[None], (_BATCH,) + y.shape)


@jax.jit
def _run(table, W1t, W2t, b2row):
    t_static, d = table.shape
    n_blocks = t_static // _BLK_T
    return pl.pallas_call(
        _ffn_block_kernel,
        grid=(n_blocks,),
        in_specs=[
            pl.BlockSpec((_BLK_T, d), lambda i: (i, 0)),
            pl.BlockSpec((d, d), lambda i: (0, 0)),
            pl.BlockSpec((d, d), lambda i: (0, 0)),
            pl.BlockSpec((1, d), lambda i: (0, 0)),
        ],
        out_specs=pl.BlockSpec((_BATCH, _BLK_T, d), lambda i: (0, i, 0)),
        out_shape=jax.ShapeDtypeStruct((_BATCH, t_static, d), jnp.float32),
        compiler_params=pltpu.CompilerParams(
            dimension_semantics=("arbitrary",),
        ),
    )(table, W1t, W2t, b2row)


def kernel(b, t, table, W1, W2, b2):
    return _run(table, W1.T, W2.T, b2.reshape(1, -1))


# speedup vs baseline: 1.1862x; 1.1141x over previous
import jax
import jax.numpy as jnp
from jax.experimental import pallas as pl
from jax.experimental.pallas import tpu as pltpu

_BATCH = 4
_BLK_T = 1024


_DIMS = (((1,), (1,)), ((), ()))


def _ffn_block_kernel(x_ref, w1_ref, w2_ref, b2_ref, out_ref):
    x = x_ref[...]
    h = jax.lax.dot_general(x, w1_ref[...], _DIMS,
                            preferred_element_type=jnp.float32)
    h = 0.5 * h * (1.0 + jax.lax.erf(h * 0.7071067811865476))
    y = jax.lax.dot_general(h, w2_ref[...], _DIMS,
                            preferred_element_type=jnp.float32)
    y = y + b2_ref[...]
    out_ref[...] = jnp.broadcast_to(y[None], (_BATCH,) + y.shape)


@jax.jit
def _run(table, W1, W2, b2row):
    t_static, d = table.shape
    n_blocks = t_static // _BLK_T
    return pl.pallas_call(
        _ffn_block_kernel,
        grid=(n_blocks,),
        in_specs=[
            pl.BlockSpec((_BLK_T, d), lambda i: (i, 0)),
            pl.BlockSpec((d, d), lambda i: (0, 0)),
            pl.BlockSpec((d, d), lambda i: (0, 0)),
            pl.BlockSpec((1, d), lambda i: (0, 0)),
        ],
        out_specs=pl.BlockSpec((_BATCH, _BLK_T, d), lambda i: (0, i, 0)),
        out_shape=jax.ShapeDtypeStruct((_BATCH, t_static, d), jnp.float32),
        compiler_params=pltpu.CompilerParams(
            dimension_semantics=("arbitrary",),
        ),
    )(table, W1, W2, b2row)


def kernel(b, t, table, W1, W2, b2):
    return _run(table, W1, W2, b2.reshape(1, -1))
